# baseline (speedup 1.0000x reference)
import functools

import jax
import jax.numpy as jnp
from jax import lax
from jax.experimental import pallas as pl
from jax.experimental.pallas import tpu as pltpu
from jax.experimental.pallas import tpu_sc as plsc

N = 10000
E = 320000
H = 128

NC = 2
NS = 16
L = 16

K1_PER_W = E // (NC * NS)
K1_CH = 80
K1_ROWS = K1_PER_W // K1_CH

K3_CHUNK = 64
K3_TPT = 312
K3_EP = 32
K3_EPB = NS * K3_TPT * K3_CHUNK
NP = 10240
ROWS_T = NP // NS
ZROWS = 128
ACC_W = H

_mesh = plsc.VectorSubcoreMesh(core_axis_name="c", subcore_axis_name="s")
_sc_params = pltpu.CompilerParams(needs_layout_passes=False)


def _silu16(x):
  return x / (1.0 + jnp.exp(-x))


def _k1a_body(idxi3, zrows, ones_in, cnt, cacc, iiv, ones_b):
  c = lax.axis_index("c")
  s = lax.axis_index("s")
  wid = s * NC + c

  def zslab(t, _):
    pltpu.sync_copy(zrows, cacc.at[pl.ds(s * ROWS_T + t * ZROWS, ZROWS)])
    return 0

  lax.fori_loop(0, ROWS_T // ZROWS, zslab, 0)
  pltpu.sync_copy(ones_in, ones_b)
  pltpu.sync_copy(idxi3.at[wid], iiv)

  plsc.subcore_barrier()

  def row(r, _):
    pltpu.sync_copy(ones_b, cacc.at[iiv.at[r]], add=True)
    return 0

  lax.fori_loop(0, K1_ROWS, row, 0)

  plsc.subcore_barrier()
  pltpu.sync_copy(cacc.at[pl.ds(s * ROWS_T, ROWS_T)],
                  cnt.at[c, pl.ds(s * ROWS_T, ROWS_T)])


_k1a = functools.partial(
    pl.kernel,
    out_type=jax.ShapeDtypeStruct((NC, NP, H), jnp.float32),
    mesh=_mesh,
    scratch_types=[
        pltpu.VMEM_SHARED((NP, H), jnp.float32),
        pltpu.VMEM((K1_ROWS, K1_CH), jnp.int32),
        pltpu.VMEM((K1_CH, H), jnp.float32),
    ],
    compiler_params=_sc_params,
)(_k1a_body)


def _k1b_body(px, py, pz, idxi3, idxj3, eax, eay, eaz, pxv, pyv, pzv, iiv,
              ijv, xo, yo, zo):
  c = lax.axis_index("c")
  s = lax.axis_index("s")
  wid = s * NC + c

  pltpu.sync_copy(px, pxv)
  pltpu.sync_copy(py, pyv)
  pltpu.sync_copy(pz, pzv)
  pltpu.sync_copy(idxi3.at[wid], iiv)
  pltpu.sync_copy(idxj3.at[wid], ijv)

  def row(r, _):
    def grp(g, _):
      sl = pl.ds(g * L, L)
      ii = iiv[r, sl]
      ij = ijv[r, sl]
      xo[r, sl] = plsc.load_gather(pxv, [ii]) - plsc.load_gather(pxv, [ij])
      yo[r, sl] = plsc.load_gather(pyv, [ii]) - plsc.load_gather(pyv, [ij])
      zo[r, sl] = plsc.load_gather(pzv, [ii]) - plsc.load_gather(pzv, [ij])
      return 0

    lax.fori_loop(0, K1_CH // L, grp, 0)
    return 0

  lax.fori_loop(0, K1_ROWS, row, 0)
  pltpu.sync_copy(xo, eax.at[wid])
  pltpu.sync_copy(yo, eay.at[wid])
  pltpu.sync_copy(zo, eaz.at[wid])


_k1b = functools.partial(
    pl.kernel,
    out_type=[
        jax.ShapeDtypeStruct((NC * NS, K1_ROWS, K1_CH), jnp.float32),
        jax.ShapeDtypeStruct((NC * NS, K1_ROWS, K1_CH), jnp.float32),
        jax.ShapeDtypeStruct((NC * NS, K1_ROWS, K1_CH), jnp.float32),
    ],
    mesh=_mesh,
    scratch_types=[
        pltpu.VMEM((N,), jnp.float32),
        pltpu.VMEM((N,), jnp.float32),
        pltpu.VMEM((N,), jnp.float32),
        pltpu.VMEM((K1_ROWS, K1_CH), jnp.int32),
        pltpu.VMEM((K1_ROWS, K1_CH), jnp.int32),
        pltpu.VMEM((K1_ROWS, K1_CH), jnp.float32),
        pltpu.VMEM((K1_ROWS, K1_CH), jnp.float32),
        pltpu.VMEM((K1_ROWS, K1_CH), jnp.float32),
    ],
    compiler_params=_sc_params,
)(_k1b_body)


def _ka_body(ne_ref, ph_ref, wit, wib, wjt, wjb, pi_ref, pj_ref):
  ne = ne_ref[...]
  ph = ph_ref[...]
  pi = jnp.dot(ne, wit[...], preferred_element_type=jnp.float32)
  pi += jnp.dot(ph, wib[...], preferred_element_type=jnp.float32)
  pj = jnp.dot(ne, wjt[...], preferred_element_type=jnp.float32)
  pj += jnp.dot(ph, wjb[...], preferred_element_type=jnp.float32)
  pi_ref[0] = pi[:, :H]
  pi_ref[1] = pi[:, H:]
  pj_ref[0] = pj[:, :H]
  pj_ref[1] = pj[:, H:]


def _ka(ne, ph, wit, wib, wjt, wjb):
  nb = 5
  rb = N // nb
  return pl.pallas_call(
      _ka_body,
      grid=(nb,),
      in_specs=[
          pl.BlockSpec((rb, H), lambda n: (n, 0)),
          pl.BlockSpec((rb, H), lambda n: (n, 0)),
          pl.BlockSpec((H, 2 * H), lambda n: (0, 0)),
          pl.BlockSpec((H, 2 * H), lambda n: (0, 0)),
          pl.BlockSpec((H, 2 * H), lambda n: (0, 0)),
          pl.BlockSpec((H, 2 * H), lambda n: (0, 0)),
      ],
      out_specs=[
          pl.BlockSpec((2, rb, H), lambda n: (0, n, 0)),
          pl.BlockSpec((2, rb, H), lambda n: (0, n, 0)),
      ],
      out_shape=[
          jax.ShapeDtypeStruct((2, N, H), jnp.float32),
          jax.ShapeDtypeStruct((2, N, H), jnp.float32),
      ],
  )(ne, ph, wit, wib, wjt, wjb)


def _k2_body(eax_ref, eay_ref, eaz_ref, w1_ref, b1, w2c, pcb, out_ref):
  w1 = w1_ref[...]
  h = (eax_ref[...] * w1[0:1] + eay_ref[...] * w1[1:2]
       + eaz_ref[...] * w1[2:3] + b1[...])
  eh = jax.nn.silu(h)
  pc = jnp.dot(eh, w2c[...], preferred_element_type=jnp.float32) + pcb[...]
  out_ref[0] = pc[:, :H].astype(jnp.bfloat16)
  out_ref[1] = pc[:, H:].astype(jnp.bfloat16)


def _k2(eax, eay, eaz, w1, b1, w2c, pcb):
  be = 4000
  nb = E // be
  return pl.pallas_call(
      _k2_body,
      grid=(nb,),
      in_specs=[
          pl.BlockSpec((be, 1), lambda e: (e, 0)),
          pl.BlockSpec((be, 1), lambda e: (e, 0)),
          pl.BlockSpec((be, 1), lambda e: (e, 0)),
          pl.BlockSpec((3, H), lambda e: (0, 0)),
          pl.BlockSpec((1, H), lambda e: (0, 0)),
          pl.BlockSpec((H, 2 * H), lambda e: (0, 0)),
          pl.BlockSpec((1, 2 * H), lambda e: (0, 0)),
      ],
      out_specs=pl.BlockSpec((2, be, H), lambda e: (0, e, 0)),
      out_shape=jax.ShapeDtypeStruct((2, E, H), jnp.bfloat16),
  )(eax, eay, eaz, w1, b1, w2c, pcb)


def _k3_real_body(idxi, idxis, idxjs, idxc, pis, pjs, pc, zrows, out, acc,
                  ij0, ij1, siv0, siv1, gi0, gi1, gj0, gj1, pcb0, pcb1, iive,
                  icve, jcve, isem0, isem1, gsem0, gsem1, psem0, psem1, ssem):
  c = lax.axis_index("c")
  s = lax.axis_index("s")
  cn = c * N

  def zslab(t, _):
    pltpu.sync_copy(zrows, acc.at[pl.ds(s * ROWS_T + t * ZROWS, ZROWS)])
    return 0

  lax.fori_loop(0, ROWS_T // ZROWS, zslab, 0)

  plsc.subcore_barrier()

  kbase = s * K3_TPT

  def wait_lin(buf, sem):
    pltpu.make_async_copy(zrows.at[pl.ds(0, buf.shape[0])], buf, sem)\
        .wait()

  def wait_idx(ij, sem):
    pltpu.make_async_copy(idxc.at[c, 0], ij, sem).wait()

  def pcslice(t):
    return pc.at[c, pl.ds((kbase + t) * K3_CHUNK, K3_CHUNK)]

  pltpu.sync_copy(idxc.at[c, kbase], ij0)
  pltpu.async_copy(idxc.at[c, kbase + 1], ij1, isem1)
  pltpu.async_copy(pcslice(0), pcb0, psem0)
  pltpu.async_copy(pcslice(1), pcb1, psem1)
  pltpu.async_copy(pis.at[ij0.at[0]], gi0, gsem0)
  pltpu.async_copy(pjs.at[ij0.at[1]], gj0, gsem0)

  def wait_pc(pcb, sem):
    pltpu.make_async_copy(pcslice(0), pcb, sem).wait()

  def phase(t, even, ij, ijn, siv, gi, gj, gin, gjn, pcb, gsem, gsemn,
            isem_c, isem_n, psem_c):
    wait_lin(gi, gsem)
    wait_lin(gj, gsem)
    for g in range(K3_CHUNK // L):
      sl = pl.ds(g * L, L)
      siv[sl] = ij[0, sl] - cn

    @pl.when(t + 2 < K3_TPT)
    def _():
      pltpu.async_copy(idxc.at[c, kbase + t + 2], ij, isem_c)

    if even:
      @pl.when(t > 0)
      def _():
        wait_lin(gi, ssem)
    else:
      wait_lin(gi, ssem)

    @pl.when(t + 1 < K3_TPT)
    def _():
      wait_idx(ijn, isem_n)
      pltpu.async_copy(pis.at[ijn.at[0]], gin, gsemn)
      pltpu.async_copy(pjs.at[ijn.at[1]], gjn, gsemn)

    wait_pc(pcb, psem_c)

    def edge(e, _):
      for g in range(H // (2 * L)):
        p1, p2 = plsc.unpack(pcb[e, pl.ds(g * 2 * L, 2 * L)],
                             format=plsc.PackFormat.INTERLEAVED)
        sl1 = pl.ds(g * 2 * L, L)
        sl2 = pl.ds(g * 2 * L + L, L)
        gi[e, sl1] = _silu16(gi[e, sl1] + gj[e, sl1] + p1)
        gi[e, sl2] = _silu16(gi[e, sl2] + gj[e, sl2] + p2)
      return 0

    lax.fori_loop(0, K3_CHUNK, edge, 0)
    pltpu.async_copy(gi, acc.at[siv], ssem, add=True)

    @pl.when(t + 2 < K3_TPT)
    def _():
      pltpu.async_copy(pcslice(t + 2), pcb, psem_c)

  def pair(p, _):
    t0 = 2 * p
    phase(t0, True, ij0, ij1, siv0, gi0, gj0, gi1, gj1, pcb0, gsem0, gsem1,
          isem0, isem1, psem0)
    phase(t0 + 1, False, ij1, ij0, siv1, gi1, gj1, gi0, gj0, pcb1, gsem1,
          gsem0, isem1, isem0, psem1)
    return 0

  lax.fori_loop(0, K3_TPT // 2, pair, 0)
  wait_lin(gi1, ssem)

  ebase = K3_EPB + s * K3_EP
  pltpu.sync_copy(idxi.at[pl.ds(ebase, K3_EP)], iive)
  pltpu.sync_copy(idxis.at[pl.ds(c * E + ebase, K3_EP)], icve)
  pltpu.sync_copy(idxjs.at[pl.ds(c * E + ebase, K3_EP)], jcve)
  gie = gi0.at[pl.ds(0, K3_EP)]
  gje = gj0.at[pl.ds(0, K3_EP)]
  pltpu.async_copy(pis.at[icve], gie, gsem0)
  pltpu.async_copy(pjs.at[jcve], gje, gsem0)
  pltpu.sync_copy(pc.at[c, pl.ds(ebase, K3_EP)], pcb0.at[pl.ds(0, K3_EP)])
  pltpu.make_async_copy(zrows.at[pl.ds(0, K3_EP)], gie, gsem0).wait()
  pltpu.make_async_copy(zrows.at[pl.ds(0, K3_EP)], gje, gsem0).wait()

  def eedge(e, _):
    for g in range(H // (2 * L)):
      p1, p2 = plsc.unpack(pcb0[e, pl.ds(g * 2 * L, 2 * L)],
                           format=plsc.PackFormat.INTERLEAVED)
      sl1 = pl.ds(g * 2 * L, L)
      sl2 = pl.ds(g * 2 * L + L, L)
      gi0[e, sl1] = _silu16(gi0[e, sl1] + gj0[e, sl1] + p1)
      gi0[e, sl2] = _silu16(gi0[e, sl2] + gj0[e, sl2] + p2)
    return 0

  lax.fori_loop(0, K3_EP, eedge, 0)
  pltpu.sync_copy(gi0.at[pl.ds(0, K3_EP)], acc.at[iive], add=True)

  plsc.subcore_barrier()
  pltpu.sync_copy(acc.at[pl.ds(s * ROWS_T, ROWS_T)],
                  out.at[c, pl.ds(s * ROWS_T, ROWS_T)])


_k3 = functools.partial(
    pl.kernel,
    out_type=jax.ShapeDtypeStruct((NC, NP, ACC_W), jnp.float32),
    mesh=_mesh,
    scratch_types=[
        pltpu.VMEM_SHARED((NP, ACC_W), jnp.float32),
        pltpu.VMEM((2, K3_CHUNK), jnp.int32),
        pltpu.VMEM((2, K3_CHUNK), jnp.int32),
        pltpu.VMEM((K3_CHUNK,), jnp.int32),
        pltpu.VMEM((K3_CHUNK,), jnp.int32),
        pltpu.VMEM((K3_CHUNK, H), jnp.float32),
        pltpu.VMEM((K3_CHUNK, H), jnp.float32),
        pltpu.VMEM((K3_CHUNK, H), jnp.float32),
        pltpu.VMEM((K3_CHUNK, H), jnp.float32),
        pltpu.VMEM((K3_CHUNK, H), jnp.bfloat16),
        pltpu.VMEM((K3_CHUNK, H), jnp.bfloat16),
        pltpu.VMEM((K3_EP,), jnp.int32),
        pltpu.VMEM((K3_EP,), jnp.int32),
        pltpu.VMEM((K3_EP,), jnp.int32),
        pltpu.SemaphoreType.DMA,
        pltpu.SemaphoreType.DMA,
        pltpu.SemaphoreType.DMA,
        pltpu.SemaphoreType.DMA,
        pltpu.SemaphoreType.DMA,
        pltpu.SemaphoreType.DMA,
        pltpu.SemaphoreType.DMA,
    ],
    compiler_params=_sc_params,
)(_k3_real_body)


def _ln(x, g, b):
  m = jnp.mean(x, axis=-1, keepdims=True)
  v = jnp.mean((x - m) ** 2, axis=-1, keepdims=True)
  return (x - m) / jnp.sqrt(v + 1e-5) * g + b


def _k4_body(acc_ref, cnt_ref, ne_ref, posp_ref, msg_w2, msg_b2, cmsg_w2,
             cmsg_b2, upd_w1, upd_b1, upd_w2, upd_b2, cupd_w1, cupd_b1,
             cupd_w2p, cupd_b2p, msg_g, msg_b, co_g, co_b, h_ref, x_ref):
  msum = acc_ref[0]
  csum = acc_ref[1]
  cnt = cnt_ref[0][:, 0:1] + cnt_ref[1][:, 0:1]
  inv = 1.0 / cnt

  agg = jnp.dot(msum * inv, msg_w2[...],
                preferred_element_type=jnp.float32) + msg_b2[...]
  agg = _ln(agg, msg_g[...], msg_b[...])
  hh = jnp.dot(jax.nn.silu(
      jnp.dot(agg, upd_w1[...], preferred_element_type=jnp.float32)
      + upd_b1[...]), upd_w2[...],
      preferred_element_type=jnp.float32) + upd_b2[...]
  h_ref[...] = hh + ne_ref[...]

  cagg = jnp.dot(csum * inv, cmsg_w2[...],
                 preferred_element_type=jnp.float32) + cmsg_b2[...]
  cagg = _ln(cagg, co_g[...], co_b[...])
  xx = jnp.dot(jax.nn.silu(
      jnp.dot(cagg, cupd_w1[...], preferred_element_type=jnp.float32)
      + cupd_b1[...]), cupd_w2p[...],
      preferred_element_type=jnp.float32) + cupd_b2p[...]
  x_ref[...] = xx + posp_ref[...]


def _k4(acc, cnt, ne, posp8, *ws):
  nb = 5
  rb = N // nb
  wspecs = []
  for w in ws:
    wspecs.append(
        pl.BlockSpec(w.shape, lambda n, r=len(w.shape): (0,) * r))
  return pl.pallas_call(
      _k4_body,
      grid=(nb,),
      in_specs=[
          pl.BlockSpec((2, rb, ACC_W), lambda n: (0, n, 0)),
          pl.BlockSpec((2, rb, H), lambda n: (0, n, 0)),
          pl.BlockSpec((rb, H), lambda n: (n, 0)),
          pl.BlockSpec((rb, 8), lambda n: (n, 0)),
      ] + wspecs,
      out_specs=[
          pl.BlockSpec((rb, H), lambda n: (n, 0)),
          pl.BlockSpec((rb, 8), lambda n: (n, 0)),
      ],
      out_shape=[
          jax.ShapeDtypeStruct((N, H), jnp.float32),
          jax.ShapeDtypeStruct((N, 8), jnp.float32),
      ],
  )(acc, cnt, ne, posp8, *ws)


def kernel(edge_index, node_embedding, pos_hidden, node_pos, edge_W1, edge_b1,
           edge_W2, edge_b2, msg_W1, msg_b1, msg_W2, msg_b2, cmsg_W1, cmsg_b1,
           cmsg_W2, cmsg_b2, upd_W1, upd_b1, upd_W2, upd_b2, cupd_W1, cupd_b1,
           cupd_W2, cupd_b2, msg_ln_g, msg_ln_b, coord_ln_g, coord_ln_b):
  f32 = jnp.float32
  idx_i = edge_index[0]
  idx_j = edge_index[1]

  w1c = jnp.concatenate([msg_W1[2 * H:3 * H], cmsg_W1[2 * H:3 * H]], axis=1)
  w2c = edge_W2 @ w1c
  pcb = (edge_b2 @ w1c + jnp.concatenate([msg_b1, cmsg_b1]))[None, :]
  blk = jnp.arange(16, dtype=jnp.int32)
  pat = jnp.stack([blk, blk + 16], axis=1).reshape(32)
  perm = jnp.concatenate([b * 32 + pat for b in range(H // 32)])
  perm2 = jnp.concatenate([perm, H + perm])
  w2c = w2c[:, perm2]
  pcb = pcb[:, perm2]

  wit = jnp.concatenate([msg_W1[0:H], cmsg_W1[0:H]], axis=1)
  wib = jnp.concatenate([msg_W1[3 * H:4 * H], cmsg_W1[3 * H:4 * H]], axis=1)
  wjt = jnp.concatenate([msg_W1[H:2 * H], cmsg_W1[H:2 * H]], axis=1)
  wjb = jnp.concatenate([msg_W1[4 * H:5 * H], cmsg_W1[4 * H:5 * H]], axis=1)

  cupd_W2p = jnp.zeros((H, 8), f32).at[:, :3].set(cupd_W2)
  cupd_b2p = jnp.zeros((1, 8), f32).at[0, :3].set(cupd_b2)
  posp8 = jnp.zeros((N, 8), f32).at[:, :3].set(node_pos)

  px = node_pos[:, 0]
  py = node_pos[:, 1]
  pz = node_pos[:, 2]
  idx_i3 = idx_i.reshape(NC * NS, K1_ROWS, K1_CH)
  idx_j3 = idx_j.reshape(NC * NS, K1_ROWS, K1_CH)
  zrows = jnp.zeros((ZROWS, H), f32)
  ones_in = jnp.zeros((K1_CH, H), f32).at[:, 0].set(1.0)
  cnt = _k1a(idx_i3, zrows, ones_in)
  eax, eay, eaz = _k1b(px, py, pz, idx_i3, idx_j3)
  eax = eax.reshape(E, 1)
  eay = eay.reshape(E, 1)
  eaz = eaz.reshape(E, 1)
  pis, pjs = _ka(node_embedding, pos_hidden, wit, wib, wjt, wjb)
  pc = _k2(eax, eay, eaz, edge_W1, edge_b1[None, :], w2c, pcb)
  pis2 = pis.reshape(2 * N, H)
  pjs2 = pjs.reshape(2 * N, H)
  idxis = jnp.concatenate([idx_i, idx_i + N])
  idxjs = jnp.concatenate([idx_j, idx_j + N])
  ij64 = jnp.stack([idx_i.reshape(-1, K3_CHUNK),
                    idx_j.reshape(-1, K3_CHUNK)], axis=1)
  idxc = jnp.stack([ij64, ij64 + N])
  acc = _k3(idx_i, idxis, idxjs, idxc, pis2, pjs2, pc, zrows)

  h, x8 = _k4(
      acc, cnt, node_embedding, posp8, msg_W2, msg_b2[None, :], cmsg_W2,
      cmsg_b2[None, :], upd_W1, upd_b1[None, :], upd_W2, upd_b2[None, :],
      cupd_W1, cupd_b1[None, :], cupd_W2p, cupd_b2p, msg_ln_g[None, :],
      msg_ln_b[None, :], coord_ln_g[None, :], coord_ln_b[None, :])
  return (h, x8[:, :3])


# speedup vs baseline: 2.3657x; 2.3657x over previous
import functools

import jax
import jax.numpy as jnp
from jax import lax
from jax.experimental import pallas as pl
from jax.experimental.pallas import tpu as pltpu
from jax.experimental.pallas import tpu_sc as plsc

N = 10000
E = 320000
H = 128

NC = 2
NS = 16
L = 16

K1_PER_W = E // (NC * NS)
K1_CH = 80
K1_ROWS = K1_PER_W // K1_CH

K3_CHUNK = 64
K3_TPT = 312
K3_EP = 32
K3_EPB = NS * K3_TPT * K3_CHUNK
NP = 10240
ROWS_T = NP // NS
ZROWS = 128
ACC_W = H

_mesh = plsc.VectorSubcoreMesh(core_axis_name="c", subcore_axis_name="s")
_sc_params = pltpu.CompilerParams(needs_layout_passes=False)


def _silu16(x):
  return x / (1.0 + jnp.exp(-x))


def _k1a_body(idxi3, zrows, ones_in, cnt, cacc, iiv, ones_b):
  c = lax.axis_index("c")
  s = lax.axis_index("s")
  wid = s * NC + c

  def zslab(t, _):
    pltpu.sync_copy(zrows, cacc.at[pl.ds(s * ROWS_T + t * ZROWS, ZROWS)])
    return 0

  lax.fori_loop(0, ROWS_T // ZROWS, zslab, 0)
  pltpu.sync_copy(ones_in, ones_b)
  pltpu.sync_copy(idxi3.at[wid], iiv)

  plsc.subcore_barrier()

  def row(r, _):
    pltpu.sync_copy(ones_b, cacc.at[iiv.at[r]], add=True)
    return 0

  lax.fori_loop(0, K1_ROWS, row, 0)

  plsc.subcore_barrier()
  pltpu.sync_copy(cacc.at[pl.ds(s * ROWS_T, ROWS_T)],
                  cnt.at[c, pl.ds(s * ROWS_T, ROWS_T)])


_k1a = functools.partial(
    pl.kernel,
    out_type=jax.ShapeDtypeStruct((NC, NP, H), jnp.float32),
    mesh=_mesh,
    scratch_types=[
        pltpu.VMEM_SHARED((NP, H), jnp.float32),
        pltpu.VMEM((K1_ROWS, K1_CH), jnp.int32),
        pltpu.VMEM((K1_CH, H), jnp.float32),
    ],
    compiler_params=_sc_params,
)(_k1a_body)


def _k1b_body(px, py, pz, idxi3, idxj3, eax, eay, eaz, pxv, pyv, pzv, iiv,
              ijv, xo, yo, zo):
  c = lax.axis_index("c")
  s = lax.axis_index("s")
  wid = s * NC + c

  pltpu.sync_copy(px, pxv)
  pltpu.sync_copy(py, pyv)
  pltpu.sync_copy(pz, pzv)
  pltpu.sync_copy(idxi3.at[wid], iiv)
  pltpu.sync_copy(idxj3.at[wid], ijv)

  def row(r, _):
    def grp(g, _):
      sl = pl.ds(g * L, L)
      ii = iiv[r, sl]
      ij = ijv[r, sl]
      xo[r, sl] = plsc.load_gather(pxv, [ii]) - plsc.load_gather(pxv, [ij])
      yo[r, sl] = plsc.load_gather(pyv, [ii]) - plsc.load_gather(pyv, [ij])
      zo[r, sl] = plsc.load_gather(pzv, [ii]) - plsc.load_gather(pzv, [ij])
      return 0

    lax.fori_loop(0, K1_CH // L, grp, 0)
    return 0

  lax.fori_loop(0, K1_ROWS, row, 0)
  pltpu.sync_copy(xo, eax.at[wid])
  pltpu.sync_copy(yo, eay.at[wid])
  pltpu.sync_copy(zo, eaz.at[wid])


_k1b = functools.partial(
    pl.kernel,
    out_type=[
        jax.ShapeDtypeStruct((NC * NS, K1_ROWS, K1_CH), jnp.float32),
        jax.ShapeDtypeStruct((NC * NS, K1_ROWS, K1_CH), jnp.float32),
        jax.ShapeDtypeStruct((NC * NS, K1_ROWS, K1_CH), jnp.float32),
    ],
    mesh=_mesh,
    scratch_types=[
        pltpu.VMEM((N,), jnp.float32),
        pltpu.VMEM((N,), jnp.float32),
        pltpu.VMEM((N,), jnp.float32),
        pltpu.VMEM((K1_ROWS, K1_CH), jnp.int32),
        pltpu.VMEM((K1_ROWS, K1_CH), jnp.int32),
        pltpu.VMEM((K1_ROWS, K1_CH), jnp.float32),
        pltpu.VMEM((K1_ROWS, K1_CH), jnp.float32),
        pltpu.VMEM((K1_ROWS, K1_CH), jnp.float32),
    ],
    compiler_params=_sc_params,
)(_k1b_body)


def _ka_body(ne_ref, ph_ref, wit, wib, wjt, wjb, pi_ref, pj_ref):
  ne = ne_ref[...]
  ph = ph_ref[...]
  pi = jnp.dot(ne, wit[...], preferred_element_type=jnp.float32)
  pi += jnp.dot(ph, wib[...], preferred_element_type=jnp.float32)
  pj = jnp.dot(ne, wjt[...], preferred_element_type=jnp.float32)
  pj += jnp.dot(ph, wjb[...], preferred_element_type=jnp.float32)
  pi_ref[0] = pi[:, :H]
  pi_ref[1] = pi[:, H:]
  pj_ref[0] = pj[:, :H]
  pj_ref[1] = pj[:, H:]


def _ka(ne, ph, wit, wib, wjt, wjb):
  nb = 5
  rb = N // nb
  return pl.pallas_call(
      _ka_body,
      grid=(nb,),
      in_specs=[
          pl.BlockSpec((rb, H), lambda n: (n, 0)),
          pl.BlockSpec((rb, H), lambda n: (n, 0)),
          pl.BlockSpec((H, 2 * H), lambda n: (0, 0)),
          pl.BlockSpec((H, 2 * H), lambda n: (0, 0)),
          pl.BlockSpec((H, 2 * H), lambda n: (0, 0)),
          pl.BlockSpec((H, 2 * H), lambda n: (0, 0)),
      ],
      out_specs=[
          pl.BlockSpec((2, rb, H), lambda n: (0, n, 0)),
          pl.BlockSpec((2, rb, H), lambda n: (0, n, 0)),
      ],
      out_shape=[
          jax.ShapeDtypeStruct((2, N, H), jnp.float32),
          jax.ShapeDtypeStruct((2, N, H), jnp.float32),
      ],
  )(ne, ph, wit, wib, wjt, wjb)


def _k2_body(eax_ref, eay_ref, eaz_ref, w1_ref, b1, w2c, pcb, out_ref):
  w1 = w1_ref[...]
  h = (eax_ref[...] * w1[0:1] + eay_ref[...] * w1[1:2]
       + eaz_ref[...] * w1[2:3] + b1[...])
  eh = jax.nn.silu(h)
  pc = jnp.dot(eh, w2c[...], preferred_element_type=jnp.float32) + pcb[...]
  out_ref[0] = pc[:, :H]
  out_ref[1] = pc[:, H:]


def _k2(eax, eay, eaz, w1, b1, w2c, pcb):
  be = 4000
  nb = E // be
  return pl.pallas_call(
      _k2_body,
      grid=(nb,),
      in_specs=[
          pl.BlockSpec((be, 1), lambda e: (e, 0)),
          pl.BlockSpec((be, 1), lambda e: (e, 0)),
          pl.BlockSpec((be, 1), lambda e: (e, 0)),
          pl.BlockSpec((3, H), lambda e: (0, 0)),
          pl.BlockSpec((1, H), lambda e: (0, 0)),
          pl.BlockSpec((H, 2 * H), lambda e: (0, 0)),
          pl.BlockSpec((1, 2 * H), lambda e: (0, 0)),
      ],
      out_specs=pl.BlockSpec((2, be, H), lambda e: (0, e, 0)),
      out_shape=jax.ShapeDtypeStruct((2, E, H), jnp.float32),
  )(eax, eay, eaz, w1, b1, w2c, pcb)


def _k3_real_body(idxi, idxis, idxjs, idxc, pis, pjs, pc, zrows, out, acc,
                  ij0, ij1, siv0, siv1, gi0, gi1, gj0, gj1, pcb0, iive,
                  icve, jcve, isem0, isem1, gsem0, gsem1, ssem):
  c = lax.axis_index("c")
  s = lax.axis_index("s")
  cn = c * N

  def zslab(t, _):
    pltpu.sync_copy(zrows, acc.at[pl.ds(s * ROWS_T + t * ZROWS, ZROWS)])
    return 0

  lax.fori_loop(0, ROWS_T // ZROWS, zslab, 0)

  plsc.subcore_barrier()

  kbase = s * K3_TPT

  def wait_lin(buf, sem):
    pltpu.make_async_copy(zrows.at[pl.ds(0, buf.shape[0])], buf, sem)\
        .wait()

  def wait_idx(ij, sem):
    pltpu.make_async_copy(idxc.at[c, 0], ij, sem).wait()

  def pcslice(t):
    return pc.at[c, pl.ds((kbase + t) * K3_CHUNK, K3_CHUNK)]

  pltpu.sync_copy(idxc.at[c, kbase], ij0)
  pltpu.async_copy(idxc.at[c, kbase + 1], ij1, isem1)
  pltpu.async_copy(pis.at[ij0.at[0]], gi0, gsem0)
  pltpu.async_copy(pjs.at[ij0.at[1]], gj0, gsem0)

  def phase(t, even, ij, ijn, siv, gi, gj, gin, gjn, gsem, gsemn,
            isem_c, isem_n):
    wait_lin(gi, gsem)
    wait_lin(gj, gsem)
    for g in range(K3_CHUNK // L):
      sl = pl.ds(g * L, L)
      siv[sl] = ij[0, sl] - cn

    @pl.when(t + 2 < K3_TPT)
    def _():
      pltpu.async_copy(idxc.at[c, kbase + t + 2], ij, isem_c)

    if even:
      @pl.when(t > 0)
      def _():
        wait_lin(gi, ssem)
    else:
      wait_lin(gi, ssem)

    @pl.when(t + 1 < K3_TPT)
    def _():
      wait_idx(ijn, isem_n)
      pltpu.async_copy(pis.at[ijn.at[0]], gin, gsemn)
      pltpu.async_copy(pjs.at[ijn.at[1]], gjn, gsemn)

    pltpu.sync_copy(pcslice(t), pcb0)

    def edge(e, _):
      for g in range(H // L):
        sl = pl.ds(g * L, L)
        x = gi[e, sl] + gj[e, sl] + pcb0[e, sl]
        gi[e, sl] = _silu16(x)
      return 0

    lax.fori_loop(0, K3_CHUNK, edge, 0)
    pltpu.async_copy(gi, acc.at[siv], ssem, add=True)

  def pair(p, _):
    t0 = 2 * p
    phase(t0, True, ij0, ij1, siv0, gi0, gj0, gi1, gj1, gsem0, gsem1,
          isem0, isem1)
    phase(t0 + 1, False, ij1, ij0, siv1, gi1, gj1, gi0, gj0, gsem1,
          gsem0, isem1, isem0)
    return 0

  lax.fori_loop(0, K3_TPT // 2, pair, 0)
  wait_lin(gi1, ssem)

  ebase = K3_EPB + s * K3_EP
  pltpu.sync_copy(idxi.at[pl.ds(ebase, K3_EP)], iive)
  pltpu.sync_copy(idxis.at[pl.ds(c * E + ebase, K3_EP)], icve)
  pltpu.sync_copy(idxjs.at[pl.ds(c * E + ebase, K3_EP)], jcve)
  gie = gi0.at[pl.ds(0, K3_EP)]
  gje = gj0.at[pl.ds(0, K3_EP)]
  pltpu.async_copy(pis.at[icve], gie, gsem0)
  pltpu.async_copy(pjs.at[jcve], gje, gsem0)
  pltpu.sync_copy(pc.at[c, pl.ds(ebase, K3_EP)], pcb0.at[pl.ds(0, K3_EP)])
  pltpu.make_async_copy(zrows.at[pl.ds(0, K3_EP)], gie, gsem0).wait()
  pltpu.make_async_copy(zrows.at[pl.ds(0, K3_EP)], gje, gsem0).wait()

  def eedge(e, _):
    for g in range(H // L):
      sl = pl.ds(g * L, L)
      x = gi0[e, sl] + gj0[e, sl] + pcb0[e, sl]
      gi0[e, sl] = _silu16(x)
    return 0

  lax.fori_loop(0, K3_EP, eedge, 0)
  pltpu.sync_copy(gi0.at[pl.ds(0, K3_EP)], acc.at[iive], add=True)

  plsc.subcore_barrier()
  pltpu.sync_copy(acc.at[pl.ds(s * ROWS_T, ROWS_T)],
                  out.at[c, pl.ds(s * ROWS_T, ROWS_T)])


_k3 = functools.partial(
    pl.kernel,
    out_type=jax.ShapeDtypeStruct((NC, NP, ACC_W), jnp.float32),
    mesh=_mesh,
    scratch_types=[
        pltpu.VMEM_SHARED((NP, ACC_W), jnp.float32),
        pltpu.VMEM((2, K3_CHUNK), jnp.int32),
        pltpu.VMEM((2, K3_CHUNK), jnp.int32),
        pltpu.VMEM((K3_CHUNK,), jnp.int32),
        pltpu.VMEM((K3_CHUNK,), jnp.int32),
        pltpu.VMEM((K3_CHUNK, H), jnp.float32),
        pltpu.VMEM((K3_CHUNK, H), jnp.float32),
        pltpu.VMEM((K3_CHUNK, H), jnp.float32),
        pltpu.VMEM((K3_CHUNK, H), jnp.float32),
        pltpu.VMEM((K3_CHUNK, H), jnp.float32),
        pltpu.VMEM((K3_EP,), jnp.int32),
        pltpu.VMEM((K3_EP,), jnp.int32),
        pltpu.VMEM((K3_EP,), jnp.int32),
        pltpu.SemaphoreType.DMA,
        pltpu.SemaphoreType.DMA,
        pltpu.SemaphoreType.DMA,
        pltpu.SemaphoreType.DMA,
        pltpu.SemaphoreType.DMA,
    ],
    compiler_params=_sc_params,
)(_k3_real_body)


def _ln(x, g, b):
  m = jnp.mean(x, axis=-1, keepdims=True)
  v = jnp.mean((x - m) ** 2, axis=-1, keepdims=True)
  return (x - m) / jnp.sqrt(v + 1e-5) * g + b


def _k4_body(acc_ref, cnt_ref, ne_ref, posp_ref, msg_w2, msg_b2, cmsg_w2,
             cmsg_b2, upd_w1, upd_b1, upd_w2, upd_b2, cupd_w1, cupd_b1,
             cupd_w2p, cupd_b2p, msg_g, msg_b, co_g, co_b, h_ref, x_ref):
  msum = acc_ref[0]
  csum = acc_ref[1]
  cnt = cnt_ref[0][:, 0:1] + cnt_ref[1][:, 0:1]
  inv = 1.0 / cnt

  agg = jnp.dot(msum * inv, msg_w2[...],
                preferred_element_type=jnp.float32) + msg_b2[...]
  agg = _ln(agg, msg_g[...], msg_b[...])
  hh = jnp.dot(jax.nn.silu(
      jnp.dot(agg, upd_w1[...], preferred_element_type=jnp.float32)
      + upd_b1[...]), upd_w2[...],
      preferred_element_type=jnp.float32) + upd_b2[...]
  h_ref[...] = hh + ne_ref[...]

  cagg = jnp.dot(csum * inv, cmsg_w2[...],
                 preferred_element_type=jnp.float32) + cmsg_b2[...]
  cagg = _ln(cagg, co_g[...], co_b[...])
  xx = jnp.dot(jax.nn.silu(
      jnp.dot(cagg, cupd_w1[...], preferred_element_type=jnp.float32)
      + cupd_b1[...]), cupd_w2p[...],
      preferred_element_type=jnp.float32) + cupd_b2p[...]
  x_ref[...] = xx + posp_ref[...]


def _k4(acc, cnt, ne, posp8, *ws):
  nb = 5
  rb = N // nb
  wspecs = []
  for w in ws:
    wspecs.append(
        pl.BlockSpec(w.shape, lambda n, r=len(w.shape): (0,) * r))
  return pl.pallas_call(
      _k4_body,
      grid=(nb,),
      in_specs=[
          pl.BlockSpec((2, rb, ACC_W), lambda n: (0, n, 0)),
          pl.BlockSpec((2, rb, H), lambda n: (0, n, 0)),
          pl.BlockSpec((rb, H), lambda n: (n, 0)),
          pl.BlockSpec((rb, 8), lambda n: (n, 0)),
      ] + wspecs,
      out_specs=[
          pl.BlockSpec((rb, H), lambda n: (n, 0)),
          pl.BlockSpec((rb, 8), lambda n: (n, 0)),
      ],
      out_shape=[
          jax.ShapeDtypeStruct((N, H), jnp.float32),
          jax.ShapeDtypeStruct((N, 8), jnp.float32),
      ],
  )(acc, cnt, ne, posp8, *ws)


def kernel(edge_index, node_embedding, pos_hidden, node_pos, edge_W1, edge_b1,
           edge_W2, edge_b2, msg_W1, msg_b1, msg_W2, msg_b2, cmsg_W1, cmsg_b1,
           cmsg_W2, cmsg_b2, upd_W1, upd_b1, upd_W2, upd_b2, cupd_W1, cupd_b1,
           cupd_W2, cupd_b2, msg_ln_g, msg_ln_b, coord_ln_g, coord_ln_b):
  f32 = jnp.float32
  idx_i = edge_index[0]
  idx_j = edge_index[1]

  w1c = jnp.concatenate([msg_W1[2 * H:3 * H], cmsg_W1[2 * H:3 * H]], axis=1)
  w2c = edge_W2 @ w1c
  pcb = (edge_b2 @ w1c + jnp.concatenate([msg_b1, cmsg_b1]))[None, :]

  wit = jnp.concatenate([msg_W1[0:H], cmsg_W1[0:H]], axis=1)
  wib = jnp.concatenate([msg_W1[3 * H:4 * H], cmsg_W1[3 * H:4 * H]], axis=1)
  wjt = jnp.concatenate([msg_W1[H:2 * H], cmsg_W1[H:2 * H]], axis=1)
  wjb = jnp.concatenate([msg_W1[4 * H:5 * H], cmsg_W1[4 * H:5 * H]], axis=1)

  cupd_W2p = jnp.zeros((H, 8), f32).at[:, :3].set(cupd_W2)
  cupd_b2p = jnp.zeros((1, 8), f32).at[0, :3].set(cupd_b2)
  posp8 = jnp.zeros((N, 8), f32).at[:, :3].set(node_pos)

  px = node_pos[:, 0]
  py = node_pos[:, 1]
  pz = node_pos[:, 2]
  idx_i3 = idx_i.reshape(NC * NS, K1_ROWS, K1_CH)
  idx_j3 = idx_j.reshape(NC * NS, K1_ROWS, K1_CH)
  zrows = jnp.zeros((ZROWS, H), f32)
  ones_in = jnp.zeros((K1_CH, H), f32).at[:, 0].set(1.0)
  cnt = _k1a(idx_i3, zrows, ones_in)
  eax, eay, eaz = _k1b(px, py, pz, idx_i3, idx_j3)
  eax = eax.reshape(E, 1)
  eay = eay.reshape(E, 1)
  eaz = eaz.reshape(E, 1)
  pis, pjs = _ka(node_embedding, pos_hidden, wit, wib, wjt, wjb)
  pc = _k2(eax, eay, eaz, edge_W1, edge_b1[None, :], w2c, pcb)
  pis2 = pis.reshape(2 * N, H)
  pjs2 = pjs.reshape(2 * N, H)
  idxis = jnp.concatenate([idx_i, idx_i + N])
  idxjs = jnp.concatenate([idx_j, idx_j + N])
  ij64 = jnp.stack([idx_i.reshape(-1, K3_CHUNK),
                    idx_j.reshape(-1, K3_CHUNK)], axis=1)
  idxc = jnp.stack([ij64, ij64 + N])
  acc = _k3(idx_i, idxis, idxjs, idxc, pis2, pjs2, pc, zrows)

  h, x8 = _k4(
      acc, cnt, node_embedding, posp8, msg_W2, msg_b2[None, :], cmsg_W2,
      cmsg_b2[None, :], upd_W1, upd_b1[None, :], upd_W2, upd_b2[None, :],
      cupd_W1, cupd_b1[None, :], cupd_W2p, cupd_b2p, msg_ln_g[None, :],
      msg_ln_b[None, :], coord_ln_g[None, :], coord_ln_b[None, :])
  return (h, x8[:, :3])
